# Initial kernel scaffold; baseline (speedup 1.0000x reference)
import jax
import jax.numpy as jnp
from jax.experimental import pallas as pl


def kernel(coordinates, adjc):
    raise NotImplementedError("write your pallas kernel here")


# speedup vs baseline: 7.7377x; 7.7377x over previous
import functools

import jax
import jax.numpy as jnp
from jax import lax
from jax.experimental import pallas as pl
from jax.experimental.pallas import tpu as pltpu
from jax.experimental.pallas import tpu_sc as plsc

N = 65536
NH = 9
E = N * NH

NC, NS, L = 2, 16, 16
NW = NC * NS
NPAIR = NW // 2
EPP = E // NPAIR
NSUB = 2
CH = EPP // NSUB
UNROLL = 8


def _sc_gather_body(lon_hbm, lat_hbm, idx_hbm, lon2_hbm, lat2_hbm,
                    table_v, idx_v, out_v):
    c = lax.axis_index("c")
    s = lax.axis_index("s")
    wid = s * NC + c
    pair = wid // 2
    half = wid % 2

    @pl.when(half == 0)
    def _():
        pltpu.sync_copy(lon_hbm, table_v)

    @pl.when(half == 1)
    def _():
        pltpu.sync_copy(lat_hbm, table_v)

    for sub in range(NSUB):
        off = pair * EPP + sub * CH
        pltpu.sync_copy(idx_hbm.at[pl.ds(off, CH)], idx_v)

        def body(i, _):
            base = i * (L * UNROLL)
            for u in range(UNROLL):
                o = base + u * L
                iv = idx_v[pl.ds(o, L)]
                out_v[pl.ds(o, L)] = plsc.load_gather(table_v, [iv])
            return 0

        lax.fori_loop(0, CH // (L * UNROLL), body, 0)

        @pl.when(half == 0)
        def _():
            pltpu.sync_copy(out_v, lon2_hbm.at[pl.ds(off, CH)])

        @pl.when(half == 1)
        def _():
            pltpu.sync_copy(out_v, lat2_hbm.at[pl.ds(off, CH)])


@jax.jit
def _sc_gather(lon, lat, idx):
    mesh = plsc.VectorSubcoreMesh(core_axis_name="c", subcore_axis_name="s",
                                  num_cores=NC, num_subcores=NS)
    f = pl.kernel(
        _sc_gather_body,
        out_type=[jax.ShapeDtypeStruct((E,), jnp.float32),
                  jax.ShapeDtypeStruct((E,), jnp.float32)],
        mesh=mesh,
        compiler_params=pltpu.CompilerParams(needs_layout_passes=False),
        scratch_types=[
            pltpu.VMEM((N,), jnp.float32),
            pltpu.VMEM((CH,), jnp.int32),
            pltpu.VMEM((CH,), jnp.float32),
        ],
        name="sc_nh_gather",
    )
    return f(lon, lat, idx)


def _tc_trig_body(lon1_ref, lat1_ref, lon2_ref, lat2_ref, dist_ref, theta_ref):
    lon1 = lon1_ref[...]
    lat1 = lat1_ref[...]
    lon2 = lon2_ref[...]
    lat2 = lat2_ref[...]
    dlon = lon2 - lon1
    coslat2 = jnp.cos(lat2)
    x = coslat2 * jnp.cos(dlon)
    y = coslat2 * jnp.sin(dlon)
    z = jnp.sin(lat2)
    coslat1 = jnp.cos(lat1)
    sinlat1 = jnp.sin(lat1)
    xr = coslat1 * x + sinlat1 * z
    zr = -sinlat1 * x + coslat1 * z
    dist_ref[...] = jnp.arctan2(jnp.sqrt(y * y + zr * zr), xr)
    theta_ref[...] = jnp.arctan2(zr, y)


_TC_ROWS = E // 128
_TC_BLOCK = 256


@jax.jit
def _tc_trig(lon1, lat1, lon2, lat2):
    shape2d = (_TC_ROWS, 128)
    bspec = pl.BlockSpec((_TC_BLOCK, 128), lambda i: (i, 0))
    return pl.pallas_call(
        _tc_trig_body,
        grid=(_TC_ROWS // _TC_BLOCK,),
        in_specs=[bspec] * 4,
        out_specs=[bspec] * 2,
        out_shape=[jax.ShapeDtypeStruct(shape2d, jnp.float32),
                   jax.ShapeDtypeStruct(shape2d, jnp.float32)],
        name="tc_rel_trig",
    )(lon1.reshape(shape2d), lat1.reshape(shape2d),
      lon2.reshape(shape2d), lat2.reshape(shape2d))


def kernel(coordinates, adjc):
    lon = coordinates[:, 0]
    lat = coordinates[:, 1]
    idx = adjc.reshape(-1)
    lon2, lat2 = _sc_gather(lon, lat, idx)
    lon1 = jnp.broadcast_to(lon[:, None], (N, NH)).reshape(-1)
    lat1 = jnp.broadcast_to(lat[:, None], (N, NH)).reshape(-1)
    dist, theta = _tc_trig(lon1, lat1, lon2, lat2)
    return jnp.stack([dist.reshape(-1), theta.reshape(-1)], axis=-1).reshape(N, NH, 2)
